# baseline (speedup 1.0000x reference)
import functools

import jax
import jax.numpy as jnp
from jax import lax
from jax.experimental import pallas as pl
from jax.experimental.pallas import tpu as pltpu
from jax.experimental.pallas import tpu_sc as plsc

N = 1_000_000
L = 16
NC = 2
NS = 16
NW = NC * NS

BLK = 65536
TC_B = 6
TC_N = 344640
SC_LEN = N - TC_N

NCH = 4
CV = 320
CVE = CV * L
VPW = CV * NCH
CPW = VPW * L
UNROLL = 8

_DNUMS = lax.GatherDimensionNumbers(
    offset_dims=(), collapsed_slice_dims=(0,), start_index_map=(0,)
)


def _gather16(table, idx):
    return lax.gather(
        table,
        idx[:, None],
        _DNUMS,
        slice_sizes=(1,),
        mode=lax.GatherScatterMode.PROMISE_IN_BOUNDS,
    )


def _make_sc_kernel():
    mesh = plsc.VectorSubcoreMesh(core_axis_name="c", subcore_axis_name="s")

    @functools.partial(
        pl.kernel,
        mesh=mesh,
        out_type=jax.ShapeDtypeStruct((SC_LEN,), jnp.float32),
        scratch_types=[
            pltpu.VMEM((CPW,), jnp.float32),
            pltpu.VMEM((CPW,), jnp.int32),
            pltpu.VMEM((CPW,), jnp.float32),
            pltpu.VMEM((L,), jnp.float32),
            pltpu.VMEM((L,), jnp.float32),
        ]
        + [pltpu.SemaphoreType.DMA] * (NCH + 2),
    )
    def k(x_hbm, sp_hbm, scale_hbm, shift_hbm, out_hbm, x_v, sp_v, o_v, tscale, tshift, *sems):
        in_sems = sems[:NCH]
        tab_sem = sems[NCH]
        out_sem = sems[NCH + 1]
        wid = lax.axis_index("s") * NC + lax.axis_index("c")
        base = TC_N + wid * CPW
        obase = wid * CPW

        pltpu.async_copy(scale_hbm, tscale, tab_sem)
        pltpu.async_copy(shift_hbm, tshift, tab_sem)
        for g in range(NCH):
            sl = pl.ds(base + g * CVE, CVE)
            vl = pl.ds(g * CVE, CVE)
            pltpu.async_copy(x_hbm.at[sl], x_v.at[vl], in_sems[g])
            pltpu.async_copy(sp_hbm.at[sl], sp_v.at[vl], in_sems[g])

        pltpu.make_async_copy(scale_hbm, tscale, tab_sem).wait()
        pltpu.make_async_copy(shift_hbm, tshift, tab_sem).wait()
        scale_vec = tscale[...]
        shift_vec = tshift[...]

        for g in range(NCH):
            sl = pl.ds(base + g * CVE, CVE)
            ol = pl.ds(obase + g * CVE, CVE)
            vl = pl.ds(g * CVE, CVE)
            pltpu.make_async_copy(x_hbm.at[sl], x_v.at[vl], in_sems[g]).wait()
            pltpu.make_async_copy(sp_hbm.at[sl], sp_v.at[vl], in_sems[g]).wait()

            def inner(j, c, g=g):
                for u in range(UNROLL):
                    vsl = pl.ds((g * CV + j * UNROLL + u) * L, L)
                    idx = sp_v[vsl]
                    xs = x_v[vsl]
                    o_v[vsl] = (
                        _gather16(scale_vec, idx) * xs + _gather16(shift_vec, idx)
                    )
                return c

            lax.fori_loop(0, CV // UNROLL, inner, 0)
            pltpu.async_copy(o_v.at[vl], out_hbm.at[ol], out_sem)

        for g in range(NCH):
            ol = pl.ds(obase + g * CVE, CVE)
            vl = pl.ds(g * CVE, CVE)
            pltpu.make_async_copy(o_v.at[vl], out_hbm.at[ol], out_sem).wait()

    return k


_sc_part = _make_sc_kernel()


def _tree_lookup(table_ref, m0, m1, m2, m3):
    lvl = [jnp.where(m0, table_ref[2 * k + 1], table_ref[2 * k]) for k in range(8)]
    lvl = [jnp.where(m1, lvl[2 * k + 1], lvl[2 * k]) for k in range(4)]
    lvl = [jnp.where(m2, lvl[2 * k + 1], lvl[2 * k]) for k in range(2)]
    return jnp.where(m3, lvl[1], lvl[0])


def _tc_body(x_ref, sp_ref, scale_ref, shift_ref, o_ref):
    s = sp_ref[...]
    x = x_ref[...]
    m0 = (s & 1) != 0
    m1 = (s & 2) != 0
    m2 = (s & 4) != 0
    m3 = (s & 8) != 0
    sc = _tree_lookup(scale_ref, m0, m1, m2, m3)
    sh = _tree_lookup(shift_ref, m0, m1, m2, m3)
    o_ref[...] = sc * x + sh


def _tc_part(x, species, scale_params, shift_params):
    return pl.pallas_call(
        _tc_body,
        grid=(TC_B,),
        in_specs=[
            pl.BlockSpec((BLK,), lambda i: (i,)),
            pl.BlockSpec((BLK,), lambda i: (i,)),
            pl.BlockSpec(memory_space=pltpu.SMEM),
            pl.BlockSpec(memory_space=pltpu.SMEM),
        ],
        out_specs=pl.BlockSpec((BLK,), lambda i: (i,)),
        out_shape=jax.ShapeDtypeStruct((N,), jnp.float32),
    )(x, species, scale_params, shift_params)


def kernel(x, species, scale_params, shift_params):
    sc_out = _sc_part(x, species, scale_params, shift_params)
    tc_out = _tc_part(x, species, scale_params, shift_params)
    return lax.dynamic_update_slice(tc_out, sc_out, (TC_N,))


# speedup vs baseline: 1.8537x; 1.8537x over previous
import functools

import jax
import jax.numpy as jnp
from jax import lax
from jax.experimental import pallas as pl
from jax.experimental.pallas import tpu as pltpu
from jax.experimental.pallas import tpu_sc as plsc

N = 1_000_000
L = 16
NC = 2
NS = 16
NW = NC * NS
NCH = 8
CV = 248
CVE = CV * L
VPW = CV * NCH
CPW = VPW * L
UNROLL = 8

_DNUMS = lax.GatherDimensionNumbers(
    offset_dims=(), collapsed_slice_dims=(0,), start_index_map=(0,)
)


def _gather16(table, idx):
    return lax.gather(
        table,
        idx[:, None],
        _DNUMS,
        slice_sizes=(1,),
        mode=lax.GatherScatterMode.PROMISE_IN_BOUNDS,
    )


def _make_kernel():
    mesh = plsc.VectorSubcoreMesh(core_axis_name="c", subcore_axis_name="s")

    @functools.partial(
        pl.kernel,
        mesh=mesh,
        out_type=jax.ShapeDtypeStruct((N,), jnp.float32),
        scratch_types=[
            pltpu.VMEM((CPW,), jnp.float32),
            pltpu.VMEM((CPW,), jnp.int32),
            pltpu.VMEM((CPW,), jnp.float32),
            pltpu.VMEM((L,), jnp.float32),
            pltpu.VMEM((L,), jnp.float32),
        ]
        + [pltpu.SemaphoreType.DMA] * (NCH + 2),
    )
    def k(x_hbm, sp_hbm, scale_hbm, shift_hbm, out_hbm, x_v, sp_v, o_v, tscale, tshift, *sems):
        in_sems = sems[:NCH]
        tab_sem = sems[NCH]
        out_sem = sems[NCH + 1]
        wid = lax.axis_index("s") * NC + lax.axis_index("c")
        base = jnp.minimum(wid * CPW, N - CPW)

        pltpu.async_copy(scale_hbm, tscale, tab_sem)
        pltpu.async_copy(shift_hbm, tshift, tab_sem)
        for g in range(NCH):
            sl = pl.ds(base + g * CVE, CVE)
            vl = pl.ds(g * CVE, CVE)
            pltpu.async_copy(x_hbm.at[sl], x_v.at[vl], in_sems[g])
            pltpu.async_copy(sp_hbm.at[sl], sp_v.at[vl], in_sems[g])

        pltpu.make_async_copy(scale_hbm, tscale, tab_sem).wait()
        pltpu.make_async_copy(shift_hbm, tshift, tab_sem).wait()
        scale_vec = tscale[...]
        shift_vec = tshift[...]

        for g in range(NCH):
            sl = pl.ds(base + g * CVE, CVE)
            vl = pl.ds(g * CVE, CVE)
            pltpu.make_async_copy(x_hbm.at[sl], x_v.at[vl], in_sems[g]).wait()
            pltpu.make_async_copy(sp_hbm.at[sl], sp_v.at[vl], in_sems[g]).wait()

            def inner(j, c, g=g):
                for u in range(UNROLL):
                    vsl = pl.ds((g * CV + j * UNROLL + u) * L, L)
                    idx = sp_v[vsl]
                    xs = x_v[vsl]
                    o_v[vsl] = (
                        _gather16(scale_vec, idx) * xs + _gather16(shift_vec, idx)
                    )
                return c

            lax.fori_loop(0, CV // UNROLL, inner, 0)
            pltpu.async_copy(o_v.at[vl], out_hbm.at[sl], out_sem)

        for g in range(NCH):
            sl = pl.ds(base + g * CVE, CVE)
            vl = pl.ds(g * CVE, CVE)
            pltpu.make_async_copy(o_v.at[vl], out_hbm.at[sl], out_sem).wait()

    return k


_scale_shift = _make_kernel()


def kernel(x, species, scale_params, shift_params):
    return _scale_shift(x, species, scale_params, shift_params)


# speedup vs baseline: 1.8551x; 1.0007x over previous
import functools

import jax
import jax.numpy as jnp
from jax import lax
from jax.experimental import pallas as pl
from jax.experimental.pallas import tpu as pltpu
from jax.experimental.pallas import tpu_sc as plsc

N = 1_000_000
L = 16
NC = 2
NS = 16
NW = NC * NS
NCH = 8
CV = 248
CVE = CV * L
VPW = CV * NCH
CPW = VPW * L
UNROLL = 8

_DNUMS = lax.GatherDimensionNumbers(
    offset_dims=(), collapsed_slice_dims=(0,), start_index_map=(0,)
)


def _gather16(table, idx):
    return lax.gather(
        table,
        idx[:, None],
        _DNUMS,
        slice_sizes=(1,),
        mode=lax.GatherScatterMode.PROMISE_IN_BOUNDS,
    )


def _make_kernel():
    mesh = plsc.VectorSubcoreMesh(core_axis_name="c", subcore_axis_name="s")

    @functools.partial(
        pl.kernel,
        mesh=mesh,
        out_type=jax.ShapeDtypeStruct((N,), jnp.float32),
        scratch_types=[
            pltpu.VMEM((CPW,), jnp.float32),
            pltpu.VMEM((CPW,), jnp.int32),
            pltpu.VMEM((CPW,), jnp.float32),
            pltpu.VMEM((L,), jnp.float32),
            pltpu.VMEM((L,), jnp.float32),
        ]
        + [pltpu.SemaphoreType.DMA] * (NCH + 2),
    )
    def k(x_hbm, sp_hbm, scale_hbm, shift_hbm, out_hbm, x_v, sp_v, o_v, tscale, tshift, *sems):
        in_sems = sems[:NCH]
        tab_sem = sems[NCH]
        out_sem = sems[NCH + 1]
        wid = lax.axis_index("s") * NC + lax.axis_index("c")
        base = jnp.minimum(wid * CPW, N - CPW)

        pltpu.async_copy(scale_hbm, tscale, tab_sem)
        pltpu.async_copy(shift_hbm, tshift, tab_sem)
        for g in range(NCH):
            sl = pl.ds(base + g * CVE, CVE)
            vl = pl.ds(g * CVE, CVE)
            pltpu.async_copy(x_hbm.at[sl], x_v.at[vl], in_sems[g])
            pltpu.async_copy(sp_hbm.at[sl], sp_v.at[vl], in_sems[g])

        pltpu.make_async_copy(scale_hbm, tscale, tab_sem).wait()
        pltpu.make_async_copy(shift_hbm, tshift, tab_sem).wait()
        scale_vec = tscale[...]
        shift_vec = tshift[...]

        def emit_out(g):
            sl = pl.ds(base + g * CVE, CVE)
            vl = pl.ds(g * CVE, CVE)
            pltpu.async_copy(o_v.at[vl], out_hbm.at[sl], out_sem)

        for g in range(NCH):
            sl = pl.ds(base + g * CVE, CVE)
            vl = pl.ds(g * CVE, CVE)
            pltpu.make_async_copy(x_hbm.at[sl], x_v.at[vl], in_sems[g]).wait()
            pltpu.make_async_copy(sp_hbm.at[sl], sp_v.at[vl], in_sems[g]).wait()

            def inner(j, c, g=g):
                for u in range(UNROLL):
                    vsl = pl.ds((g * CV + j * UNROLL + u) * L, L)
                    idx = sp_v[vsl]
                    xs = x_v[vsl]
                    o_v[vsl] = (
                        _gather16(scale_vec, idx) * xs + _gather16(shift_vec, idx)
                    )
                return c

            lax.fori_loop(0, CV // UNROLL, inner, 0)
            if g >= 1:
                emit_out(g - 1)

        plsc.subcore_barrier()
        emit_out(NCH - 1)

        for g in range(NCH):
            sl = pl.ds(base + g * CVE, CVE)
            vl = pl.ds(g * CVE, CVE)
            pltpu.make_async_copy(o_v.at[vl], out_hbm.at[sl], out_sem).wait()

    return k


_scale_shift = _make_kernel()


def kernel(x, species, scale_params, shift_params):
    return _scale_shift(x, species, scale_params, shift_params)
